# baseline (speedup 1.0000x reference)
import functools

import jax
import jax.numpy as jnp
from jax import lax
from jax.experimental import pallas as pl
from jax.experimental.pallas import tpu as pltpu
from jax.experimental.pallas import tpu_sc as plsc

N = 10000
E = 160000
D = 256
H = 256
G = 64

NC = 2
NS = 16
CH = 128
EG = 16

HALF = N // 2
TRASH = HALF
ACC_ROWS = HALF + 8
EPT = E // NS
NFULL = EPT // CH
TAIL = EPT - NFULL * CH
IPT = 320
LAST_D = ACC_ROWS - (NS - 1) * IPT
FCH = 64

QR = N // 4
TRASH_P = QR
ACCP_ROWS = QR + 8
IPT_Q = 160
LAST_Q = QR - (NS - 1) * IPT_Q


@functools.cache
def _sc_kernels():
  mesh = plsc.VectorSubcoreMesh(core_axis_name="c", subcore_axis_name="s")

  @functools.partial(
      pl.kernel,
      out_type=(jax.ShapeDtypeStruct((ACC_ROWS,), jnp.float32),
                jax.ShapeDtypeStruct((ACC_ROWS,), jnp.float32)),
      mesh=mesh,
      scratch_types=[
          pltpu.VMEM((CH,), jnp.int32),
          pltpu.VMEM((CH,), jnp.int32),
          pltpu.VMEM((CH,), jnp.float32),
          pltpu.VMEM((16,), jnp.int32),
          pltpu.VMEM((16,), jnp.int32),
          pltpu.VMEM((16,), jnp.float32),
          pltpu.VMEM((IPT,), jnp.float32),
          pltpu.VMEM((IPT,), jnp.float32),
          pltpu.VMEM_SHARED((ACC_ROWS,), jnp.float32),
      ],
  )
  def sc_degree(dst_hbm, deg0_hbm, deg1_hbm, didx, tidx, ones,
                didx_t, tidx_t, ones_t, ones_init, dbuf, accum):
    c = lax.axis_index("c")
    s = lax.axis_index("s")
    base = c * HALF
    one = jnp.ones((16,), jnp.float32)
    for j in range(CH // 16):
      ones[pl.ds(j * 16, 16)] = one
    ones_t[...] = one
    for j in range(IPT // 16):
      ones_init[pl.ds(j * 16, 16)] = one

    @pl.when(s < NS - 1)
    def _():
      pltpu.sync_copy(ones_init.at[pl.ds(0, IPT)],
                      accum.at[pl.ds(s * IPT, IPT)])

    @pl.when(s == NS - 1)
    def _():
      pltpu.sync_copy(ones_init.at[pl.ds(0, LAST_D)],
                      accum.at[pl.ds((NS - 1) * IPT, LAST_D)])

    plsc.subcore_barrier()

    e0 = s * EPT

    def _edges(off, n, di, ti, vals):
      pltpu.sync_copy(dst_hbm.at[pl.ds(off, n)], di)
      for j in range(n // 16):
        d = di[pl.ds(j * 16, 16)]
        loc = d - base
        ok = (loc >= 0) & (loc < HALF)
        ti[pl.ds(j * 16, 16)] = jnp.where(ok, loc, TRASH)
      pltpu.sync_copy(vals, accum.at[ti], add=True)

    def body(g, carry):
      _edges(e0 + g * CH, CH, didx, tidx, ones)
      return carry

    lax.fori_loop(0, NFULL, body, None)
    _edges(e0 + NFULL * CH, TAIL, didx_t, tidx_t, ones_t)

    plsc.subcore_barrier()

    def _writeout(r0, n):
      pltpu.sync_copy(accum.at[pl.ds(r0, n)], dbuf.at[pl.ds(0, n)])

      @pl.when(c == 0)
      def _():
        pltpu.sync_copy(dbuf.at[pl.ds(0, n)], deg0_hbm.at[pl.ds(r0, n)])

      @pl.when(c == 1)
      def _():
        pltpu.sync_copy(dbuf.at[pl.ds(0, n)], deg1_hbm.at[pl.ds(r0, n)])

    @pl.when(s < NS - 1)
    def _():
      _writeout(s * IPT, IPT)

    @pl.when(s == NS - 1)
    def _():
      _writeout((NS - 1) * IPT, LAST_D)

  @functools.partial(
      pl.kernel,
      out_type=jax.ShapeDtypeStruct((N * H,), jnp.float32),
      mesh=mesh,
      scratch_types=[
          pltpu.VMEM((CH,), jnp.int32),
          pltpu.VMEM((CH,), jnp.int32),
          pltpu.VMEM((CH,), jnp.int32),
          pltpu.VMEM((CH, H), jnp.float32),
          pltpu.VMEM((2 * CH, CH), jnp.int32),
          pltpu.VMEM((16,), jnp.int32),
          pltpu.VMEM((16,), jnp.int32),
          pltpu.VMEM((16,), jnp.int32),
          pltpu.VMEM((16, H), jnp.float32),
          pltpu.VMEM((FCH * H,), jnp.float32),
          pltpu.VMEM_SHARED((ACCP_ROWS * H,), jnp.float32),
          pltpu.SemaphoreType.DMA,
          pltpu.SemaphoreType.DMA,
      ],
  )
  def sc_propagate(z_hbm, zflat_hbm, src_hbm, dst_hbm, out_hbm,
                   sidx, didx, tidx, rows, idxb, sidx_t, didx_t, tidx_t,
                   rows_t, fbuf, accum, gsem, ssem):
    c = lax.axis_index("c")
    s = lax.axis_index("s")
    e0 = s * EPT
    iotas = [lax.iota(jnp.int32, 16) + 16 * k for k in range(H // 16)]

    for p in range(2):
      base = c * HALF + p * QR

      def _acc_init(r0, n, base=base):
        done = 0
        while done < n:
          sz = min(FCH, n - done)
          pltpu.sync_copy(
              zflat_hbm.at[pl.ds((base + r0 + done) * H, sz * H)],
              fbuf.at[pl.ds(0, sz * H)])
          pltpu.sync_copy(fbuf.at[pl.ds(0, sz * H)],
                          accum.at[pl.ds((r0 + done) * H, sz * H)])
          done += sz

      @pl.when(s < NS - 1)
      def _():
        _acc_init(s * IPT_Q, IPT_Q)

      @pl.when(s == NS - 1)
      def _():
        _acc_init((NS - 1) * IPT_Q, LAST_Q)

      plsc.subcore_barrier()

      def _edges(off, n, si, di, ti, rw, base=base):
        pltpu.sync_copy(src_hbm.at[pl.ds(off, n)], si)
        pltpu.sync_copy(dst_hbm.at[pl.ds(off, n)], di)
        for j in range(n // 16):
          d = di[pl.ds(j * 16, 16)]
          loc = d - base
          ok = (loc >= 0) & (loc < QR)
          ti[pl.ds(j * 16, 16)] = jnp.where(ok, loc, TRASH_P) * H
        pltpu.async_copy(z_hbm.at[si], rw, gsem).wait()

        def grp(g16, carry):
          tv = ti[pl.ds(g16 * EG, EG)]
          for el in range(EG):
            bv = tv[el]
            for k in range(H // 16):
              idxb[2 * el + (k // 8), pl.ds(16 * (k % 8), 16)] = bv + iotas[k]
          cps = []
          for el in range(EG):
            e = g16 * EG + el
            for hh in range(2):
              cps.append(pltpu.async_copy(
                  rw.at[e, pl.ds(128 * hh, 128)],
                  accum.at[idxb.at[2 * el + hh]],
                  ssem, add=True))
          for cp in cps:
            cp.wait()
          return carry

        lax.fori_loop(0, n // EG, grp, None)

      def body(g, carry):
        _edges(e0 + g * CH, CH, sidx, didx, tidx, rows)
        return carry

      lax.fori_loop(0, NFULL, body, None)
      _edges(e0 + NFULL * CH, TAIL, sidx_t, didx_t, tidx_t, rows_t)

      plsc.subcore_barrier()

      def _writeout(r0, n, base=base):
        done = 0
        while done < n:
          sz = min(FCH, n - done)
          pltpu.sync_copy(accum.at[pl.ds((r0 + done) * H, sz * H)],
                          fbuf.at[pl.ds(0, sz * H)])
          pltpu.sync_copy(
              fbuf.at[pl.ds(0, sz * H)],
              out_hbm.at[pl.ds((base + r0 + done) * H, sz * H)])
          done += sz

      @pl.when(s < NS - 1)
      def _():
        _writeout(s * IPT_Q, IPT_Q)

      @pl.when(s == NS - 1)
      def _():
        _writeout((NS - 1) * IPT_Q, LAST_Q)

      plsc.subcore_barrier()

  return sc_degree, sc_propagate


BLK = 1000
GRID = N // BLK


def _tc_first_body(x_ref, w_ref, deg_ref, z_ref):
    dis = lax.rsqrt(deg_ref[...])
    z_ref[...] = jnp.dot(x_ref[...], w_ref[...],
                         preferred_element_type=jnp.float32) * dis


def _tc_first(x, W, deg2):
    return pl.pallas_call(
        _tc_first_body,
        grid=(GRID,),
        in_specs=[
            pl.BlockSpec((BLK, D), lambda i: (i, 0)),
            pl.BlockSpec((D, H), lambda i: (0, 0)),
            pl.BlockSpec((BLK, 1), lambda i: (i, 0)),
        ],
        out_specs=pl.BlockSpec((BLK, H), lambda i: (i, 0)),
        out_shape=jax.ShapeDtypeStruct((N, H), jnp.float32),
    )(x, W, deg2)


def _tc_mid_body(s_ref, b_ref, w_ref, deg_ref, o_ref):
    dis = lax.rsqrt(deg_ref[...])
    h = jnp.maximum(s_ref[...] * dis + b_ref[...], 0.0)
    o_ref[...] = jnp.dot(h, w_ref[...],
                         preferred_element_type=jnp.float32) * dis


def _tc_mid(sacc, b, W, deg2):
    return pl.pallas_call(
        _tc_mid_body,
        grid=(GRID,),
        in_specs=[
            pl.BlockSpec((BLK, H), lambda i: (i, 0)),
            pl.BlockSpec((1, H), lambda i: (0, 0)),
            pl.BlockSpec((H, H), lambda i: (0, 0)),
            pl.BlockSpec((BLK, 1), lambda i: (i, 0)),
        ],
        out_specs=pl.BlockSpec((BLK, H), lambda i: (i, 0)),
        out_shape=jax.ShapeDtypeStruct((N, H), jnp.float32),
    )(sacc, b, W, deg2)


def _tc_pool_body(s_ref, b_ref, deg_ref, batch_ref, wl_ref, bl_ref,
                  out_ref, acc_ref):
    i = pl.program_id(0)
    dis = lax.rsqrt(deg_ref[...])
    h = s_ref[...] * dis + b_ref[...]
    oh = (batch_ref[...] == lax.broadcasted_iota(jnp.int32, (BLK, G), 1))
    part = lax.dot_general(oh.astype(jnp.float32), h,
                           (((0,), (0,)), ((), ())),
                           preferred_element_type=jnp.float32)

    @pl.when(i == 0)
    def _():
        acc_ref[...] = part

    @pl.when(i > 0)
    def _():
        acc_ref[...] += part

    @pl.when(i == GRID - 1)
    def _():
        out_ref[...] = jnp.dot(acc_ref[...], wl_ref[...],
                               preferred_element_type=jnp.float32) + bl_ref[...]


def _tc_pool(sacc, b, deg2, batch2, Wl, bl2):
    return pl.pallas_call(
        _tc_pool_body,
        grid=(GRID,),
        in_specs=[
            pl.BlockSpec((BLK, H), lambda i: (i, 0)),
            pl.BlockSpec((1, H), lambda i: (0, 0)),
            pl.BlockSpec((BLK, 1), lambda i: (i, 0)),
            pl.BlockSpec((BLK, 1), lambda i: (i, 0)),
            pl.BlockSpec((H, 1), lambda i: (0, 0)),
            pl.BlockSpec((1, 1), lambda i: (0, 0)),
        ],
        out_specs=pl.BlockSpec((G, 1), lambda i: (0, 0)),
        out_shape=jax.ShapeDtypeStruct((G, 1), jnp.float32),
        scratch_shapes=[pltpu.VMEM((G, H), jnp.float32)],
    )(sacc, b, deg2, batch2, Wl, bl2)


def kernel(x, edge_index, batch, W1, b1, W2, b2, W3, b3, Wl, bl):
    sc_degree, sc_propagate = _sc_kernels()
    src = edge_index[0]
    dst = edge_index[1]
    deg0, deg1 = sc_degree(dst)
    deg2 = jnp.concatenate([deg0[:HALF], deg1[:HALF]]).reshape(N, 1)
    z1 = _tc_first(x, W1, deg2)
    s1 = sc_propagate(z1, z1.reshape(N * H), src, dst).reshape(N, H)
    z2 = _tc_mid(s1, b1.reshape(1, H), W2, deg2)
    s2 = sc_propagate(z2, z2.reshape(N * H), src, dst).reshape(N, H)
    z3 = _tc_mid(s2, b2.reshape(1, H), W3, deg2)
    s3 = sc_propagate(z3, z3.reshape(N * H), src, dst).reshape(N, H)
    return _tc_pool(s3, b3.reshape(1, H), deg2, batch.reshape(N, 1),
                    Wl, bl.reshape(1, 1))


# speedup vs baseline: 1.0816x; 1.0816x over previous
import functools

import jax
import jax.numpy as jnp
from jax import lax
from jax.experimental import pallas as pl
from jax.experimental.pallas import tpu as pltpu
from jax.experimental.pallas import tpu_sc as plsc

N = 10000
E = 160000
D = 256
H = 256
G = 64

NC = 2
NS = 16
CH = 128
EG = 16

HALF = N // 2
TRASH = HALF
ACC_ROWS = HALF + 8
EPT = E // NS
NFULL = EPT // CH
TAIL = EPT - NFULL * CH
IPT = 320
LAST_D = ACC_ROWS - (NS - 1) * IPT
FCH = 64

QR = N // 4
TRASH_P = QR
ACCP_ROWS = QR + 8
IPT_Q = 160
LAST_Q = QR - (NS - 1) * IPT_Q


@functools.cache
def _sc_kernels():
  mesh = plsc.VectorSubcoreMesh(core_axis_name="c", subcore_axis_name="s")

  @functools.partial(
      pl.kernel,
      out_type=(jax.ShapeDtypeStruct((ACC_ROWS,), jnp.float32),
                jax.ShapeDtypeStruct((ACC_ROWS,), jnp.float32)),
      mesh=mesh,
      scratch_types=[
          pltpu.VMEM((CH,), jnp.int32),
          pltpu.VMEM((CH,), jnp.int32),
          pltpu.VMEM((CH,), jnp.float32),
          pltpu.VMEM((16,), jnp.int32),
          pltpu.VMEM((16,), jnp.int32),
          pltpu.VMEM((16,), jnp.float32),
          pltpu.VMEM((IPT,), jnp.float32),
          pltpu.VMEM((IPT,), jnp.float32),
          pltpu.VMEM_SHARED((ACC_ROWS,), jnp.float32),
      ],
  )
  def sc_degree(dst_hbm, deg0_hbm, deg1_hbm, didx, tidx, ones,
                didx_t, tidx_t, ones_t, ones_init, dbuf, accum):
    c = lax.axis_index("c")
    s = lax.axis_index("s")
    base = c * HALF
    one = jnp.ones((16,), jnp.float32)
    for j in range(CH // 16):
      ones[pl.ds(j * 16, 16)] = one
    ones_t[...] = one
    for j in range(IPT // 16):
      ones_init[pl.ds(j * 16, 16)] = one

    @pl.when(s < NS - 1)
    def _():
      pltpu.sync_copy(ones_init.at[pl.ds(0, IPT)],
                      accum.at[pl.ds(s * IPT, IPT)])

    @pl.when(s == NS - 1)
    def _():
      pltpu.sync_copy(ones_init.at[pl.ds(0, LAST_D)],
                      accum.at[pl.ds((NS - 1) * IPT, LAST_D)])

    plsc.subcore_barrier()

    e0 = s * EPT

    def _edges(off, n, di, ti, vals):
      pltpu.sync_copy(dst_hbm.at[pl.ds(off, n)], di)
      for j in range(n // 16):
        d = di[pl.ds(j * 16, 16)]
        loc = d - base
        ok = (loc >= 0) & (loc < HALF)
        ti[pl.ds(j * 16, 16)] = jnp.where(ok, loc, TRASH)
      pltpu.sync_copy(vals, accum.at[ti], add=True)

    def body(g, carry):
      _edges(e0 + g * CH, CH, didx, tidx, ones)
      return carry

    lax.fori_loop(0, NFULL, body, None)
    _edges(e0 + NFULL * CH, TAIL, didx_t, tidx_t, ones_t)

    plsc.subcore_barrier()

    def _writeout(r0, n):
      pltpu.sync_copy(accum.at[pl.ds(r0, n)], dbuf.at[pl.ds(0, n)])

      @pl.when(c == 0)
      def _():
        pltpu.sync_copy(dbuf.at[pl.ds(0, n)], deg0_hbm.at[pl.ds(r0, n)])

      @pl.when(c == 1)
      def _():
        pltpu.sync_copy(dbuf.at[pl.ds(0, n)], deg1_hbm.at[pl.ds(r0, n)])

    @pl.when(s < NS - 1)
    def _():
      _writeout(s * IPT, IPT)

    @pl.when(s == NS - 1)
    def _():
      _writeout((NS - 1) * IPT, LAST_D)

  @functools.partial(
      pl.kernel,
      out_type=jax.ShapeDtypeStruct((N * H,), jnp.float32),
      mesh=mesh,
      scratch_types=[
          pltpu.VMEM((CH,), jnp.int32),
          pltpu.VMEM((CH,), jnp.int32),
          pltpu.VMEM((CH,), jnp.int32),
          pltpu.VMEM((CH, H), jnp.float32),
          pltpu.VMEM((CH,), jnp.int32),
          pltpu.VMEM((16,), jnp.int32),
          pltpu.VMEM((16,), jnp.int32),
          pltpu.VMEM((16,), jnp.int32),
          pltpu.VMEM((16, H), jnp.float32),
          pltpu.VMEM((FCH * H,), jnp.float32),
          pltpu.VMEM_SHARED((ACCP_ROWS * H,), jnp.float32),
          pltpu.SemaphoreType.DMA,
          pltpu.SemaphoreType.DMA,
      ],
  )
  def sc_propagate(z_hbm, zflat_hbm, src_hbm, dst_hbm, out_hbm,
                   sidx, didx, tidx, rows, idxb, sidx_t, didx_t, tidx_t,
                   rows_t, fbuf, accum, gsem, ssem):
    c = lax.axis_index("c")
    s = lax.axis_index("s")
    e0 = s * EPT
    for k in range(CH // 16):
      idxb[pl.ds(16 * k, 16)] = lax.iota(jnp.int32, 16) + 16 * k

    for p in range(2):
      base = c * HALF + p * QR

      def _acc_init(r0, n, base=base):
        done = 0
        while done < n:
          sz = min(FCH, n - done)
          pltpu.sync_copy(
              zflat_hbm.at[pl.ds((base + r0 + done) * H, sz * H)],
              fbuf.at[pl.ds(0, sz * H)])
          pltpu.sync_copy(fbuf.at[pl.ds(0, sz * H)],
                          accum.at[pl.ds((r0 + done) * H, sz * H)])
          done += sz

      @pl.when(s < NS - 1)
      def _():
        _acc_init(s * IPT_Q, IPT_Q)

      @pl.when(s == NS - 1)
      def _():
        _acc_init((NS - 1) * IPT_Q, LAST_Q)

      plsc.subcore_barrier()

      def _edges(off, n, si, di, ti, rw, base=base):
        pltpu.sync_copy(src_hbm.at[pl.ds(off, n)], si)
        pltpu.sync_copy(dst_hbm.at[pl.ds(off, n)], di)
        for j in range(n // 16):
          d = di[pl.ds(j * 16, 16)]
          loc = d - base
          ok = (loc >= 0) & (loc < QR)
          ti[pl.ds(j * 16, 16)] = jnp.where(ok, loc, TRASH_P) * H
        pltpu.async_copy(z_hbm.at[si], rw, gsem).wait()

        def grp(g16, carry):
          tv = ti[pl.ds(g16 * EG, EG)]
          cps = []
          for el in range(EG):
            e = g16 * EG + el
            bv = tv[el]
            for hh in range(2):
              off = pl.multiple_of(bv + 128 * hh, 128)
              cps.append(pltpu.async_copy(
                  rw.at[e, pl.ds(128 * hh, 128)],
                  accum.at[pl.ds(off, 128)].at[idxb],
                  ssem, add=True))
          for cp in cps:
            cp.wait()
          return carry

        lax.fori_loop(0, n // EG, grp, None)

      def body(g, carry):
        _edges(e0 + g * CH, CH, sidx, didx, tidx, rows)
        return carry

      lax.fori_loop(0, NFULL, body, None)
      _edges(e0 + NFULL * CH, TAIL, sidx_t, didx_t, tidx_t, rows_t)

      plsc.subcore_barrier()

      def _writeout(r0, n, base=base):
        done = 0
        while done < n:
          sz = min(FCH, n - done)
          pltpu.sync_copy(accum.at[pl.ds((r0 + done) * H, sz * H)],
                          fbuf.at[pl.ds(0, sz * H)])
          pltpu.sync_copy(
              fbuf.at[pl.ds(0, sz * H)],
              out_hbm.at[pl.ds((base + r0 + done) * H, sz * H)])
          done += sz

      @pl.when(s < NS - 1)
      def _():
        _writeout(s * IPT_Q, IPT_Q)

      @pl.when(s == NS - 1)
      def _():
        _writeout((NS - 1) * IPT_Q, LAST_Q)

      plsc.subcore_barrier()

  return sc_degree, sc_propagate


BLK = 1000
GRID = N // BLK


def _tc_first_body(x_ref, w_ref, deg_ref, z_ref):
    dis = lax.rsqrt(deg_ref[...])
    z_ref[...] = jnp.dot(x_ref[...], w_ref[...],
                         preferred_element_type=jnp.float32) * dis


def _tc_first(x, W, deg2):
    return pl.pallas_call(
        _tc_first_body,
        grid=(GRID,),
        in_specs=[
            pl.BlockSpec((BLK, D), lambda i: (i, 0)),
            pl.BlockSpec((D, H), lambda i: (0, 0)),
            pl.BlockSpec((BLK, 1), lambda i: (i, 0)),
        ],
        out_specs=pl.BlockSpec((BLK, H), lambda i: (i, 0)),
        out_shape=jax.ShapeDtypeStruct((N, H), jnp.float32),
    )(x, W, deg2)


def _tc_mid_body(s_ref, b_ref, w_ref, deg_ref, o_ref):
    dis = lax.rsqrt(deg_ref[...])
    h = jnp.maximum(s_ref[...] * dis + b_ref[...], 0.0)
    o_ref[...] = jnp.dot(h, w_ref[...],
                         preferred_element_type=jnp.float32) * dis


def _tc_mid(sacc, b, W, deg2):
    return pl.pallas_call(
        _tc_mid_body,
        grid=(GRID,),
        in_specs=[
            pl.BlockSpec((BLK, H), lambda i: (i, 0)),
            pl.BlockSpec((1, H), lambda i: (0, 0)),
            pl.BlockSpec((H, H), lambda i: (0, 0)),
            pl.BlockSpec((BLK, 1), lambda i: (i, 0)),
        ],
        out_specs=pl.BlockSpec((BLK, H), lambda i: (i, 0)),
        out_shape=jax.ShapeDtypeStruct((N, H), jnp.float32),
    )(sacc, b, W, deg2)


def _tc_pool_body(s_ref, b_ref, deg_ref, batch_ref, wl_ref, bl_ref,
                  out_ref, acc_ref):
    i = pl.program_id(0)
    dis = lax.rsqrt(deg_ref[...])
    h = s_ref[...] * dis + b_ref[...]
    oh = (batch_ref[...] == lax.broadcasted_iota(jnp.int32, (BLK, G), 1))
    part = lax.dot_general(oh.astype(jnp.float32), h,
                           (((0,), (0,)), ((), ())),
                           preferred_element_type=jnp.float32)

    @pl.when(i == 0)
    def _():
        acc_ref[...] = part

    @pl.when(i > 0)
    def _():
        acc_ref[...] += part

    @pl.when(i == GRID - 1)
    def _():
        out_ref[...] = jnp.dot(acc_ref[...], wl_ref[...],
                               preferred_element_type=jnp.float32) + bl_ref[...]


def _tc_pool(sacc, b, deg2, batch2, Wl, bl2):
    return pl.pallas_call(
        _tc_pool_body,
        grid=(GRID,),
        in_specs=[
            pl.BlockSpec((BLK, H), lambda i: (i, 0)),
            pl.BlockSpec((1, H), lambda i: (0, 0)),
            pl.BlockSpec((BLK, 1), lambda i: (i, 0)),
            pl.BlockSpec((BLK, 1), lambda i: (i, 0)),
            pl.BlockSpec((H, 1), lambda i: (0, 0)),
            pl.BlockSpec((1, 1), lambda i: (0, 0)),
        ],
        out_specs=pl.BlockSpec((G, 1), lambda i: (0, 0)),
        out_shape=jax.ShapeDtypeStruct((G, 1), jnp.float32),
        scratch_shapes=[pltpu.VMEM((G, H), jnp.float32)],
    )(sacc, b, deg2, batch2, Wl, bl2)


def kernel(x, edge_index, batch, W1, b1, W2, b2, W3, b3, Wl, bl):
    sc_degree, sc_propagate = _sc_kernels()
    src = edge_index[0]
    dst = edge_index[1]
    deg0, deg1 = sc_degree(dst)
    deg2 = jnp.concatenate([deg0[:HALF], deg1[:HALF]]).reshape(N, 1)
    z1 = _tc_first(x, W1, deg2)
    s1 = sc_propagate(z1, z1.reshape(N * H), src, dst).reshape(N, H)
    z2 = _tc_mid(s1, b1.reshape(1, H), W2, deg2)
    s2 = sc_propagate(z2, z2.reshape(N * H), src, dst).reshape(N, H)
    z3 = _tc_mid(s2, b2.reshape(1, H), W3, deg2)
    s3 = sc_propagate(z3, z3.reshape(N * H), src, dst).reshape(N, H)
    return _tc_pool(s3, b3.reshape(1, H), deg2, batch.reshape(N, 1),
                    Wl, bl.reshape(1, 1))


# speedup vs baseline: 1.3030x; 1.2047x over previous
import functools

import jax
import jax.numpy as jnp
from jax import lax
from jax.experimental import pallas as pl
from jax.experimental.pallas import tpu as pltpu
from jax.experimental.pallas import tpu_sc as plsc

N = 10000
E = 160000
D = 256
H = 256
G = 64

NC = 2
NS = 16
CH = 128
EG = 16

HALF = N // 2
TRASH = HALF
ACC_ROWS = HALF + 8
EPT = E // NS
NFULL = EPT // CH
TAIL = EPT - NFULL * CH
IPT = 320
LAST_D = ACC_ROWS - (NS - 1) * IPT
FCH = 64

QR = N // 4
TRASH_P = QR
ACCP_ROWS = QR + 8
IPT_Q = 160
LAST_Q = QR - (NS - 1) * IPT_Q


@functools.cache
def _sc_kernels():
  mesh = plsc.VectorSubcoreMesh(core_axis_name="c", subcore_axis_name="s")

  @functools.partial(
      pl.kernel,
      out_type=(jax.ShapeDtypeStruct((ACC_ROWS,), jnp.float32),
                jax.ShapeDtypeStruct((ACC_ROWS,), jnp.float32)),
      mesh=mesh,
      scratch_types=[
          pltpu.VMEM((CH,), jnp.int32),
          pltpu.VMEM((CH,), jnp.int32),
          pltpu.VMEM((CH,), jnp.float32),
          pltpu.VMEM((16,), jnp.int32),
          pltpu.VMEM((16,), jnp.int32),
          pltpu.VMEM((16,), jnp.float32),
          pltpu.VMEM((IPT,), jnp.float32),
          pltpu.VMEM((IPT,), jnp.float32),
          pltpu.VMEM_SHARED((ACC_ROWS,), jnp.float32),
      ],
  )
  def sc_degree(dst_hbm, deg0_hbm, deg1_hbm, didx, tidx, ones,
                didx_t, tidx_t, ones_t, ones_init, dbuf, accum):
    c = lax.axis_index("c")
    s = lax.axis_index("s")
    base = c * HALF
    one = jnp.ones((16,), jnp.float32)
    for j in range(CH // 16):
      ones[pl.ds(j * 16, 16)] = one
    ones_t[...] = one
    for j in range(IPT // 16):
      ones_init[pl.ds(j * 16, 16)] = one

    @pl.when(s < NS - 1)
    def _():
      pltpu.sync_copy(ones_init.at[pl.ds(0, IPT)],
                      accum.at[pl.ds(s * IPT, IPT)])

    @pl.when(s == NS - 1)
    def _():
      pltpu.sync_copy(ones_init.at[pl.ds(0, LAST_D)],
                      accum.at[pl.ds((NS - 1) * IPT, LAST_D)])

    plsc.subcore_barrier()

    e0 = s * EPT

    def _edges(off, n, di, ti, vals):
      pltpu.sync_copy(dst_hbm.at[pl.ds(off, n)], di)
      for j in range(n // 16):
        d = di[pl.ds(j * 16, 16)]
        loc = d - base
        ok = (loc >= 0) & (loc < HALF)
        ti[pl.ds(j * 16, 16)] = jnp.where(ok, loc, TRASH)
      pltpu.sync_copy(vals, accum.at[ti], add=True)

    def body(g, carry):
      _edges(e0 + g * CH, CH, didx, tidx, ones)
      return carry

    lax.fori_loop(0, NFULL, body, None)
    _edges(e0 + NFULL * CH, TAIL, didx_t, tidx_t, ones_t)

    plsc.subcore_barrier()

    def _writeout(r0, n):
      pltpu.sync_copy(accum.at[pl.ds(r0, n)], dbuf.at[pl.ds(0, n)])

      @pl.when(c == 0)
      def _():
        pltpu.sync_copy(dbuf.at[pl.ds(0, n)], deg0_hbm.at[pl.ds(r0, n)])

      @pl.when(c == 1)
      def _():
        pltpu.sync_copy(dbuf.at[pl.ds(0, n)], deg1_hbm.at[pl.ds(r0, n)])

    @pl.when(s < NS - 1)
    def _():
      _writeout(s * IPT, IPT)

    @pl.when(s == NS - 1)
    def _():
      _writeout((NS - 1) * IPT, LAST_D)

  @functools.partial(
      pl.kernel,
      out_type=jax.ShapeDtypeStruct((N * H,), jnp.float32),
      mesh=mesh,
      scratch_types=[
          pltpu.VMEM((CH,), jnp.int32),
          pltpu.VMEM((CH,), jnp.int32),
          pltpu.VMEM((CH,), jnp.int32),
          pltpu.VMEM((CH, H), jnp.float32),
          pltpu.VMEM((CH,), jnp.int32),
          pltpu.VMEM((CH,), jnp.int32),
          pltpu.VMEM((CH,), jnp.int32),
          pltpu.VMEM((CH, H), jnp.float32),
          pltpu.VMEM((CH,), jnp.int32),
          pltpu.VMEM((16,), jnp.int32),
          pltpu.VMEM((16,), jnp.int32),
          pltpu.VMEM((16,), jnp.int32),
          pltpu.VMEM((16, H), jnp.float32),
          pltpu.VMEM((FCH * H,), jnp.float32),
          pltpu.VMEM_SHARED((ACCP_ROWS * H,), jnp.float32),
          pltpu.SemaphoreType.DMA,
          pltpu.SemaphoreType.DMA,
          pltpu.SemaphoreType.DMA,
      ],
  )
  def sc_propagate(z_hbm, zflat_hbm, src_hbm, dst_hbm, out_hbm,
                   sidx, didx, tidx, rows, sidx2, didx2, tidx2, rows2,
                   idxb, sidx_t, didx_t, tidx_t,
                   rows_t, fbuf, accum, gsem, gsem2, ssem):
    c = lax.axis_index("c")
    s = lax.axis_index("s")
    e0 = s * EPT
    for k in range(CH // 16):
      idxb[pl.ds(16 * k, 16)] = lax.iota(jnp.int32, 16) + 16 * k

    for p in range(2):
      base = c * HALF + p * QR

      def _acc_init(r0, n, base=base):
        done = 0
        while done < n:
          sz = min(FCH, n - done)
          pltpu.sync_copy(
              zflat_hbm.at[pl.ds((base + r0 + done) * H, sz * H)],
              fbuf.at[pl.ds(0, sz * H)])
          pltpu.sync_copy(fbuf.at[pl.ds(0, sz * H)],
                          accum.at[pl.ds((r0 + done) * H, sz * H)])
          done += sz

      @pl.when(s < NS - 1)
      def _():
        _acc_init(s * IPT_Q, IPT_Q)

      @pl.when(s == NS - 1)
      def _():
        _acc_init((NS - 1) * IPT_Q, LAST_Q)

      plsc.subcore_barrier()

      def _issue(off, si, di, ti, rw, sem, base=base):
        pltpu.sync_copy(src_hbm.at[pl.ds(off, CH)], si)
        pltpu.sync_copy(dst_hbm.at[pl.ds(off, CH)], di)
        for j in range(CH // 16):
          d = di[pl.ds(j * 16, 16)]
          loc = d - base
          ok = (loc >= 0) & (loc < QR)
          ti[pl.ds(j * 16, 16)] = jnp.where(ok, loc, TRASH_P) * H
        pltpu.async_copy(z_hbm.at[si], rw, sem)

      def _fire(ti, rw, g16, e_base):
        tv = ti[pl.ds(g16 * EG, EG)]
        cps = []
        for el in range(EG):
          e = e_base + el
          bv = tv[el]
          for hh in range(2):
            off = pl.multiple_of(bv + 128 * hh, 128)
            cps.append(pltpu.async_copy(
                rw.at[e, pl.ds(128 * hh, 128)],
                accum.at[pl.ds(off, 128)].at[idxb],
                ssem, add=True))
        return cps

      def _scatter(ti, rw):
        prev = []
        for g16 in range(CH // EG):
          cur = _fire(ti, rw, g16, g16 * EG)
          for cp in prev:
            cp.wait()
          prev = cur
        for cp in prev:
          cp.wait()

      _issue(e0, sidx, didx, tidx, rows, gsem)

      def body(g, carry):
        _issue(e0 + (2 * g + 1) * CH, sidx2, didx2, tidx2, rows2, gsem2)
        pltpu.make_async_copy(z_hbm.at[sidx], rows, gsem).wait()
        _scatter(tidx, rows)

        @pl.when(g < NFULL // 2 - 1)
        def _():
          _issue(e0 + (2 * g + 2) * CH, sidx, didx, tidx, rows, gsem)

        pltpu.make_async_copy(z_hbm.at[sidx2], rows2, gsem2).wait()
        _scatter(tidx2, rows2)
        return carry

      lax.fori_loop(0, NFULL // 2, body, None)

      toff = e0 + NFULL * CH
      pltpu.sync_copy(src_hbm.at[pl.ds(toff, TAIL)], sidx_t)
      pltpu.sync_copy(dst_hbm.at[pl.ds(toff, TAIL)], didx_t)
      d = didx_t[...]
      loc = d - base
      ok = (loc >= 0) & (loc < QR)
      tidx_t[...] = jnp.where(ok, loc, TRASH_P) * H
      pltpu.async_copy(z_hbm.at[sidx_t], rows_t, gsem).wait()
      tv = tidx_t[...]
      cps = []
      for el in range(TAIL):
        bv = tv[el]
        for hh in range(2):
          off2 = pl.multiple_of(bv + 128 * hh, 128)
          cps.append(pltpu.async_copy(
              rows_t.at[el, pl.ds(128 * hh, 128)],
              accum.at[pl.ds(off2, 128)].at[idxb],
              ssem, add=True))
      for cp in cps:
        cp.wait()

      plsc.subcore_barrier()

      def _writeout(r0, n, base=base):
        done = 0
        while done < n:
          sz = min(FCH, n - done)
          pltpu.sync_copy(accum.at[pl.ds((r0 + done) * H, sz * H)],
                          fbuf.at[pl.ds(0, sz * H)])
          pltpu.sync_copy(
              fbuf.at[pl.ds(0, sz * H)],
              out_hbm.at[pl.ds((base + r0 + done) * H, sz * H)])
          done += sz

      @pl.when(s < NS - 1)
      def _():
        _writeout(s * IPT_Q, IPT_Q)

      @pl.when(s == NS - 1)
      def _():
        _writeout((NS - 1) * IPT_Q, LAST_Q)

      plsc.subcore_barrier()

  return sc_degree, sc_propagate


BLK = 1000
GRID = N // BLK


def _tc_first_body(x_ref, w_ref, deg_ref, z_ref):
    dis = lax.rsqrt(deg_ref[...])
    z_ref[...] = jnp.dot(x_ref[...], w_ref[...],
                         preferred_element_type=jnp.float32) * dis


def _tc_first(x, W, deg2):
    return pl.pallas_call(
        _tc_first_body,
        grid=(GRID,),
        in_specs=[
            pl.BlockSpec((BLK, D), lambda i: (i, 0)),
            pl.BlockSpec((D, H), lambda i: (0, 0)),
            pl.BlockSpec((BLK, 1), lambda i: (i, 0)),
        ],
        out_specs=pl.BlockSpec((BLK, H), lambda i: (i, 0)),
        out_shape=jax.ShapeDtypeStruct((N, H), jnp.float32),
    )(x, W, deg2)


def _tc_mid_body(s_ref, b_ref, w_ref, deg_ref, o_ref):
    dis = lax.rsqrt(deg_ref[...])
    h = jnp.maximum(s_ref[...] * dis + b_ref[...], 0.0)
    o_ref[...] = jnp.dot(h, w_ref[...],
                         preferred_element_type=jnp.float32) * dis


def _tc_mid(sacc, b, W, deg2):
    return pl.pallas_call(
        _tc_mid_body,
        grid=(GRID,),
        in_specs=[
            pl.BlockSpec((BLK, H), lambda i: (i, 0)),
            pl.BlockSpec((1, H), lambda i: (0, 0)),
            pl.BlockSpec((H, H), lambda i: (0, 0)),
            pl.BlockSpec((BLK, 1), lambda i: (i, 0)),
        ],
        out_specs=pl.BlockSpec((BLK, H), lambda i: (i, 0)),
        out_shape=jax.ShapeDtypeStruct((N, H), jnp.float32),
    )(sacc, b, W, deg2)


def _tc_pool_body(s_ref, b_ref, deg_ref, batch_ref, wl_ref, bl_ref,
                  out_ref, acc_ref):
    i = pl.program_id(0)
    dis = lax.rsqrt(deg_ref[...])
    h = s_ref[...] * dis + b_ref[...]
    oh = (batch_ref[...] == lax.broadcasted_iota(jnp.int32, (BLK, G), 1))
    part = lax.dot_general(oh.astype(jnp.float32), h,
                           (((0,), (0,)), ((), ())),
                           preferred_element_type=jnp.float32)

    @pl.when(i == 0)
    def _():
        acc_ref[...] = part

    @pl.when(i > 0)
    def _():
        acc_ref[...] += part

    @pl.when(i == GRID - 1)
    def _():
        out_ref[...] = jnp.dot(acc_ref[...], wl_ref[...],
                               preferred_element_type=jnp.float32) + bl_ref[...]


def _tc_pool(sacc, b, deg2, batch2, Wl, bl2):
    return pl.pallas_call(
        _tc_pool_body,
        grid=(GRID,),
        in_specs=[
            pl.BlockSpec((BLK, H), lambda i: (i, 0)),
            pl.BlockSpec((1, H), lambda i: (0, 0)),
            pl.BlockSpec((BLK, 1), lambda i: (i, 0)),
            pl.BlockSpec((BLK, 1), lambda i: (i, 0)),
            pl.BlockSpec((H, 1), lambda i: (0, 0)),
            pl.BlockSpec((1, 1), lambda i: (0, 0)),
        ],
        out_specs=pl.BlockSpec((G, 1), lambda i: (0, 0)),
        out_shape=jax.ShapeDtypeStruct((G, 1), jnp.float32),
        scratch_shapes=[pltpu.VMEM((G, H), jnp.float32)],
    )(sacc, b, deg2, batch2, Wl, bl2)


def kernel(x, edge_index, batch, W1, b1, W2, b2, W3, b3, Wl, bl):
    sc_degree, sc_propagate = _sc_kernels()
    src = edge_index[0]
    dst = edge_index[1]
    deg0, deg1 = sc_degree(dst)
    deg2 = jnp.concatenate([deg0[:HALF], deg1[:HALF]]).reshape(N, 1)
    z1 = _tc_first(x, W1, deg2)
    s1 = sc_propagate(z1, z1.reshape(N * H), src, dst).reshape(N, H)
    z2 = _tc_mid(s1, b1.reshape(1, H), W2, deg2)
    s2 = sc_propagate(z2, z2.reshape(N * H), src, dst).reshape(N, H)
    z3 = _tc_mid(s2, b2.reshape(1, H), W3, deg2)
    s3 = sc_propagate(z3, z3.reshape(N * H), src, dst).reshape(N, H)
    return _tc_pool(s3, b3.reshape(1, H), deg2, batch.reshape(N, 1),
                    Wl, bl.reshape(1, 1))
